# Initial kernel scaffold; baseline (speedup 1.0000x reference)
import jax
import jax.numpy as jnp
from jax.experimental import pallas as pl


def kernel(x, table):
    raise NotImplementedError("write your pallas kernel here")


# speedup vs baseline: 2.0400x; 2.0400x over previous
import functools

import jax
import jax.numpy as jnp
from jax import lax
from jax.experimental import pallas as pl
from jax.experimental.pallas import tpu as pltpu
from jax.experimental.pallas import tpu_sc as plsc

B, L, F = 4096, 200, 16
EMBED_DIM = 32
OUT_F = F - 1 + EMBED_DIM
N = B * L

NC, NS, LANES = 2, 16, 16
NW = NC * NS
RPW = N // NW
C = 512
NCHUNK = RPW // C
IDX_MINOR = 128
NSUB = C // IDX_MINOR


def _sc_body(x_hbm, table_hbm, out_hbm, x_v, keys_v, emb_v, out_v, sem):
    wid = lax.axis_index("s") * NC + lax.axis_index("c")
    lane = lax.iota(jnp.int32, LANES)

    def chunk_body(ci, _):
        row0 = wid * RPW + ci * C

        pltpu.sync_copy(x_hbm.at[pl.ds(row0 * F, C * F)], x_v)

        for j in range(C // LANES):
            kf = plsc.load_gather(x_v, [(lane + j * LANES) * F + (F - 1)])
            keys_v[j // (IDX_MINOR // LANES),
                   pl.ds((j % (IDX_MINOR // LANES)) * LANES, LANES)] = (
                kf.astype(jnp.int32))

        copies = [
            pltpu.make_async_copy(
                table_hbm.at[keys_v.at[j]],
                emb_v.at[pl.ds(j * IDX_MINOR, IDX_MINOR), :],
                sem,
            )
            for j in range(NSUB)
        ]
        for cp in copies:
            cp.start()
        for cp in copies:
            cp.wait()

        def row_body(r, _):
            o = r * OUT_F
            out_v[pl.ds(o, LANES)] = x_v[pl.ds(r * F, LANES)]
            out_v[pl.ds(o + F - 1, LANES)] = emb_v[r, pl.ds(0, LANES)]
            out_v[pl.ds(o + F - 1 + LANES, LANES)] = emb_v[r, pl.ds(LANES, LANES)]
            return 0

        lax.fori_loop(0, C, row_body, 0, unroll=4)

        pltpu.sync_copy(out_v, out_hbm.at[pl.ds(row0 * OUT_F, C * OUT_F)])
        return 0

    lax.fori_loop(0, NCHUNK, chunk_body, 0)


@jax.jit
def _sc_call(x2d, table):
    mesh = plsc.VectorSubcoreMesh(core_axis_name="c", subcore_axis_name="s")
    return pl.kernel(
        _sc_body,
        out_type=jax.ShapeDtypeStruct((N * OUT_F,), jnp.float32),
        mesh=mesh,
        compiler_params=pltpu.CompilerParams(
            needs_layout_passes=False, use_tc_tiling_on_sc=False),
        scratch_types=[
            pltpu.VMEM((C * F,), jnp.float32),
            pltpu.VMEM((NSUB, IDX_MINOR), jnp.int32),
            pltpu.VMEM((C, EMBED_DIM), jnp.float32),
            pltpu.VMEM((C * OUT_F,), jnp.float32),
            pltpu.SemaphoreType.DMA,
        ],
    )(x2d, table)


def kernel(x, table):
    x1d = x.reshape(N * F)
    out = _sc_call(x1d, table)
    return out.reshape(B, L, OUT_F)


# speedup vs baseline: 2.1300x; 1.0441x over previous
import functools

import jax
import jax.numpy as jnp
from jax import lax
from jax.experimental import pallas as pl
from jax.experimental.pallas import tpu as pltpu
from jax.experimental.pallas import tpu_sc as plsc

B, L, F = 4096, 200, 16
EMBED_DIM = 32
OUT_F = F - 1 + EMBED_DIM
N = B * L

NC, NS, LANES = 2, 16, 16
NW = NC * NS
RPW = N // NW
C = 512
NCHUNK = RPW // C
IDX_MINOR = 128
NSUB = C // IDX_MINOR


def _sc_body(x_hbm, table_hbm, out_hbm,
             x_v, keys_v, emb_v, out_v, sem_in, sem_emb, sem_out):
    wid = lax.axis_index("s") * NC + lax.axis_index("c")
    lane = lax.iota(jnp.int32, LANES)
    base = wid * RPW

    def in_copy(ci, b):
        row0 = base + ci * C
        return pltpu.make_async_copy(
            x_hbm.at[pl.ds(row0 * F, C * F)], x_v.at[b], sem_in.at[b])

    def out_copy(ci, b):
        row0 = base + ci * C
        return pltpu.make_async_copy(
            out_v.at[b], out_hbm.at[pl.ds(row0 * OUT_F, C * OUT_F)],
            sem_out.at[b])

    in_copy(0, 0).start()

    @pl.loop(0, NCHUNK // 2)
    def chunk_pair(i):
        for par in range(2):
            b = par
            ci = i * 2 + par

            in_copy(ci, b).wait()

            @pl.when(ci + 1 < NCHUNK)
            def _():
                in_copy(ci + 1, 1 - b).start()

            for j in range(C // LANES):
                kf = plsc.load_gather(
                    x_v.at[b], [(lane + j * LANES) * F + (F - 1)])
                keys_v[b, j // (IDX_MINOR // LANES),
                       pl.ds((j % (IDX_MINOR // LANES)) * LANES, LANES)] = (
                    kf.astype(jnp.int32))

            copies = [
                pltpu.make_async_copy(
                    table_hbm.at[keys_v.at[b, j]],
                    emb_v.at[b, pl.ds(j * IDX_MINOR, IDX_MINOR), :],
                    sem_emb.at[b],
                )
                for j in range(NSUB)
            ]
            for cp in copies:
                cp.start()
            for cp in copies:
                cp.wait()

            @pl.when(ci >= 2)
            def _():
                out_copy(ci - 2, b).wait()

            @pl.loop(0, C, unroll=8)
            def row_body(r):
                o = r * OUT_F
                out_v[b, pl.ds(o, LANES)] = x_v[b, pl.ds(r * F, LANES)]
                out_v[b, pl.ds(o + F - 1, LANES)] = emb_v[b, r, pl.ds(0, LANES)]
                out_v[b, pl.ds(o + F - 1 + LANES, LANES)] = (
                    emb_v[b, r, pl.ds(LANES, LANES)])

            out_copy(ci, b).start()

    out_copy(NCHUNK - 2, 0).wait()
    out_copy(NCHUNK - 1, 1).wait()


@jax.jit
def _sc_call(x1d, table):
    mesh = plsc.VectorSubcoreMesh(core_axis_name="c", subcore_axis_name="s")
    return pl.kernel(
        _sc_body,
        out_type=jax.ShapeDtypeStruct((N * OUT_F,), jnp.float32),
        mesh=mesh,
        scratch_types=[
            pltpu.VMEM((2, C * F), jnp.float32),
            pltpu.VMEM((2, NSUB, IDX_MINOR), jnp.int32),
            pltpu.VMEM((2, C, EMBED_DIM), jnp.float32),
            pltpu.VMEM((2, C * OUT_F), jnp.float32),
            pltpu.SemaphoreType.DMA((2,)),
            pltpu.SemaphoreType.DMA((2,)),
            pltpu.SemaphoreType.DMA((2,)),
        ],
        compiler_params=pltpu.CompilerParams(
            needs_layout_passes=False, use_tc_tiling_on_sc=False),
    )(x1d, table)


def kernel(x, table):
    x1d = x.reshape(N * F)
    out = _sc_call(x1d, table)
    return out.reshape(B, L, OUT_F)


# speedup vs baseline: 3.2745x; 1.5373x over previous
import functools

import jax
import jax.numpy as jnp
from jax import lax
from jax.experimental import pallas as pl
from jax.experimental.pallas import tpu as pltpu
from jax.experimental.pallas import tpu_sc as plsc

B, L, F = 4096, 200, 16
EMBED_DIM = 32
OUT_F = F - 1 + EMBED_DIM

NC, NS, LANES = 2, 16, 16
NW = NC * NS
LT = L // 8
NLB = 4
NBLK = L // NLB
CR = NLB * 128


def _sc_body(x_hbm, table_hbm, out_hbm, x_v, keys_v, emb_v, out_v, sem):
    wid = lax.axis_index("s") * NC + lax.axis_index("c")
    lane = lax.iota(jnp.int32, LANES)

    def blk_body(bi, _):
        l0 = bi * NLB
        lt = l0 // 8
        s0 = l0 % 8

        pltpu.sync_copy(x_hbm.at[pl.ds(l0, NLB), :, wid, :, :], x_v)

        for li in range(NLB):
            for k in range(8):
                kf = x_v[li, 1, 7, pl.ds(k * LANES, LANES)]
                keys_v[li, pl.ds(k * LANES, LANES)] = kf.astype(jnp.int32)

        copies = [
            pltpu.make_async_copy(
                table_hbm.at[keys_v.at[li]],
                emb_v.at[pl.ds(li * 128, 128), :],
                sem,
            )
            for li in range(NLB)
        ]
        for cp in copies:
            cp.start()
        for cp in copies:
            cp.wait()

        @pl.loop(0, F - 1)
        def timing_plane(c):
            tf = c // 8
            s = c % 8
            for li in range(NLB):
                for k in range(8):
                    out_v[c, li, pl.ds(k * LANES, LANES)] = (
                        x_v[li, tf, s, pl.ds(k * LANES, LANES)])

        @pl.loop(F - 1, OUT_F)
        def emb_plane(c):
            e = c - (F - 1)
            for li in range(NLB):
                for k in range(8):
                    rows = li * 128 + k * LANES + lane
                    vals = plsc.load_gather(
                        emb_v, [rows, jnp.full((LANES,), e, jnp.int32)])
                    out_v[c, li, pl.ds(k * LANES, LANES)] = vals

        pltpu.sync_copy(out_v, out_hbm.at[:, lt, wid, pl.ds(s0, NLB), :])
        return 0

    lax.fori_loop(0, NBLK, blk_body, 0)


@jax.jit
def _sc_call(x5, table):
    mesh = plsc.VectorSubcoreMesh(core_axis_name="c", subcore_axis_name="s")
    return pl.kernel(
        _sc_body,
        out_type=jax.ShapeDtypeStruct((OUT_F, LT, NW, 8, 128), jnp.float32),
        mesh=mesh,
        scratch_types=[
            pltpu.VMEM((NLB, 2, 8, 128), jnp.float32),
            pltpu.VMEM((NLB, 128), jnp.int32),
            pltpu.VMEM((CR, EMBED_DIM), jnp.float32),
            pltpu.VMEM((OUT_F, NLB, 128), jnp.float32),
            pltpu.SemaphoreType.DMA,
        ],
        compiler_params=pltpu.CompilerParams(
            needs_layout_passes=False, use_tc_tiling_on_sc=False),
    )(x5, table)


def kernel(x, table):
    x5 = x.transpose(1, 2, 0).reshape(L, 2, 8, 32, 128).transpose(0, 1, 3, 2, 4)
    out5 = _sc_call(x5, table)
    out = out5.transpose(0, 1, 3, 2, 4).reshape(OUT_F, L, B).transpose(2, 1, 0)
    return out


# speedup vs baseline: 3.5462x; 1.0830x over previous
import functools

import jax
import jax.numpy as jnp
from jax import lax
from jax.experimental import pallas as pl
from jax.experimental.pallas import tpu as pltpu
from jax.experimental.pallas import tpu_sc as plsc

B, L, F = 4096, 200, 16
EMBED_DIM = 32
OUT_F = F - 1 + EMBED_DIM

NC, NS, LANES = 2, 16, 16
NW = NC * NS
LT = L // 8
NLB = 4
NBLK = L // NLB
CR = NLB * 128


def _sc_body(x_hbm, table_hbm, out_hbm,
             x_v, keys_v, emb_v, out_v, sem_in, sem_emb, sem_out):
    wid = lax.axis_index("s") * NC + lax.axis_index("c")
    lane = lax.iota(jnp.int32, LANES)

    def in_copy(bi, d):
        return pltpu.make_async_copy(
            x_hbm.at[pl.ds(bi * NLB, NLB), :, wid, :, :], x_v.at[d],
            sem_in.at[d])

    def out_copy(bi, d):
        l0 = bi * NLB
        return pltpu.make_async_copy(
            out_v.at[d],
            out_hbm.at[:, l0 // 8, wid, pl.ds(l0 % 8, NLB), :],
            sem_out.at[d])

    in_copy(0, 0).start()

    @pl.loop(0, NBLK // 2)
    def blk_pair(i):
        for d in range(2):
            bi = i * 2 + d

            in_copy(bi, d).wait()

            @pl.when(bi + 1 < NBLK)
            def _():
                in_copy(bi + 1, 1 - d).start()

            for li in range(NLB):
                for k in range(8):
                    kf = x_v[d, li, 1, 7, pl.ds(k * LANES, LANES)]
                    keys_v[d, li, pl.ds(k * LANES, LANES)] = (
                        kf.astype(jnp.int32))

            copies = [
                pltpu.make_async_copy(
                    table_hbm.at[keys_v.at[d, li]],
                    emb_v.at[d, pl.ds(li * 128, 128), :],
                    sem_emb.at[d],
                )
                for li in range(NLB)
            ]
            for cp in copies:
                cp.start()
            for cp in copies:
                cp.wait()

            @pl.when(bi >= 2)
            def _():
                out_copy(bi - 2, d).wait()

            @pl.loop(0, F - 1, unroll=5)
            def timing_plane(c):
                tf = c // 8
                s = c % 8
                for li in range(NLB):
                    for k in range(8):
                        out_v[d, c, li, pl.ds(k * LANES, LANES)] = (
                            x_v[d, li, tf, s, pl.ds(k * LANES, LANES)])

            @pl.loop(F - 1, OUT_F, unroll=4)
            def emb_plane(c):
                e = c - (F - 1)
                for li in range(NLB):
                    for k in range(8):
                        rows = li * 128 + k * LANES + lane
                        vals = plsc.load_gather(
                            emb_v.at[d],
                            [rows, jnp.full((LANES,), e, jnp.int32)])
                        out_v[d, c, li, pl.ds(k * LANES, LANES)] = vals

            out_copy(bi, d).start()

    out_copy(NBLK - 2, 0).wait()
    out_copy(NBLK - 1, 1).wait()


@jax.jit
def _sc_call(x5, table):
    mesh = plsc.VectorSubcoreMesh(core_axis_name="c", subcore_axis_name="s")
    return pl.kernel(
        _sc_body,
        out_type=jax.ShapeDtypeStruct((OUT_F, LT, NW, 8, 128), jnp.float32),
        mesh=mesh,
        scratch_types=[
            pltpu.VMEM((2, NLB, 2, 8, 128), jnp.float32),
            pltpu.VMEM((2, NLB, 128), jnp.int32),
            pltpu.VMEM((2, CR, EMBED_DIM), jnp.float32),
            pltpu.VMEM((2, OUT_F, NLB, 128), jnp.float32),
            pltpu.SemaphoreType.DMA((2,)),
            pltpu.SemaphoreType.DMA((2,)),
            pltpu.SemaphoreType.DMA((2,)),
        ],
        compiler_params=pltpu.CompilerParams(
            needs_layout_passes=False, use_tc_tiling_on_sc=False),
    )(x5, table)


def kernel(x, table):
    x5 = x.transpose(1, 2, 0).reshape(L, 2, 8, 32, 128).transpose(0, 1, 3, 2, 4)
    out5 = _sc_call(x5, table)
    out = out5.transpose(0, 1, 3, 2, 4).reshape(OUT_F, L, B).transpose(2, 1, 0)
    return out


# speedup vs baseline: 6.0549x; 1.7074x over previous
import functools

import jax
import jax.numpy as jnp
from jax import lax
from jax.experimental import pallas as pl
from jax.experimental.pallas import tpu as pltpu
from jax.experimental.pallas import tpu_sc as plsc

B, L, F = 4096, 200, 16
EMBED_DIM = 32
OUT_F = F - 1 + EMBED_DIM

NC, NS, LANES = 2, 16, 16
NW = NC * NS
LT = L // 8
NLB = 4
NBLK = L // NLB
CR = NLB * 128


def _sc_body(x_hbm, table_hbm, out_hbm,
             x_v, keys_v, emb_v, out_v, sem_in, sem_emb, sem_out):
    wid = lax.axis_index("s") * NC + lax.axis_index("c")
    lane = lax.iota(jnp.int32, LANES)

    def in_copy(bi, d):
        return pltpu.make_async_copy(
            x_hbm.at[pl.ds(bi * NLB, NLB), :, wid, :, :], x_v.at[d],
            sem_in.at[d])

    def out_copy(bi, d):
        l0 = bi * NLB
        return pltpu.make_async_copy(
            out_v.at[d],
            out_hbm.at[:, l0 // 8, wid, pl.ds(l0 % 8, NLB), :],
            sem_out.at[d])

    in_copy(0, 0).start()

    @pl.loop(0, NBLK // 2)
    def blk_pair(i):
        for d in range(2):
            bi = i * 2 + d

            in_copy(bi, d).wait()

            @pl.when(bi + 1 < NBLK)
            def _():
                in_copy(bi + 1, 1 - d).start()

            for li in range(NLB):
                for k in range(8):
                    kf = x_v[d, li, 1, 7, pl.ds(k * LANES, LANES)]
                    keys_v[d, li, pl.ds(k * LANES, LANES)] = (
                        kf.astype(jnp.int32))

            copies = [
                pltpu.make_async_copy(
                    table_hbm.at[keys_v.at[d, li]],
                    emb_v.at[d, pl.ds(li * 128, 128), :],
                    sem_emb.at[d],
                )
                for li in range(NLB)
            ]
            for cp in copies:
                cp.start()
            for cp in copies:
                cp.wait()

            @pl.when(bi >= 2)
            def _():
                out_copy(bi - 2, d).wait()

            @plsc.parallel_loop(0, F - 1, unroll=5)
            def timing_plane(c):
                tf = c // 8
                s = c % 8
                for li in range(NLB):
                    for k in range(8):
                        out_v[d, c, li, pl.ds(k * LANES, LANES)] = (
                            x_v[d, li, tf, s, pl.ds(k * LANES, LANES)])

            @plsc.parallel_loop(F - 1, OUT_F, unroll=4)
            def emb_plane(c):
                e = c - (F - 1)
                for li in range(NLB):
                    for k in range(8):
                        rows = li * 128 + k * LANES + lane
                        vals = plsc.load_gather(
                            emb_v.at[d],
                            [rows, jnp.full((LANES,), e, jnp.int32)])
                        out_v[d, c, li, pl.ds(k * LANES, LANES)] = vals

            out_copy(bi, d).start()

    out_copy(NBLK - 2, 0).wait()
    out_copy(NBLK - 1, 1).wait()


@jax.jit
def _sc_call(x5, table):
    mesh = plsc.VectorSubcoreMesh(core_axis_name="c", subcore_axis_name="s")
    return pl.kernel(
        _sc_body,
        out_type=jax.ShapeDtypeStruct((OUT_F, LT, NW, 8, 128), jnp.float32),
        mesh=mesh,
        scratch_types=[
            pltpu.VMEM((2, NLB, 2, 8, 128), jnp.float32),
            pltpu.VMEM((2, NLB, 128), jnp.int32),
            pltpu.VMEM((2, CR, EMBED_DIM), jnp.float32),
            pltpu.VMEM((2, OUT_F, NLB, 128), jnp.float32),
            pltpu.SemaphoreType.DMA((2,)),
            pltpu.SemaphoreType.DMA((2,)),
            pltpu.SemaphoreType.DMA((2,)),
        ],
        compiler_params=pltpu.CompilerParams(
            needs_layout_passes=False, use_tc_tiling_on_sc=False),
    )(x5, table)


def kernel(x, table):
    x5 = x.transpose(1, 2, 0).reshape(L, 2, 8, 32, 128).transpose(0, 1, 3, 2, 4)
    out5 = _sc_call(x5, table)
    out = out5.transpose(0, 1, 3, 2, 4).reshape(OUT_F, L, B).transpose(2, 1, 0)
    return out
